# baseline (speedup 1.0000x reference)
import functools

import jax
import jax.numpy as jnp
from jax import lax
from jax.experimental import pallas as pl
from jax.experimental.pallas import tpu as pltpu
from jax.experimental.pallas import tpu_sc as plsc

BATCH = 16384
DIM = 64
NUM_REL = 1000
L = 16
NW = 32
BPW = BATCH // NW
CH = 128
NCH = BPW // CH
GPC = CH // L

_mesh = plsc.VectorSubcoreMesh(core_axis_name="c", subcore_axis_name="s")


@functools.partial(
    pl.kernel,
    out_type=jax.ShapeDtypeStruct((BATCH,), jnp.float32),
    mesh=_mesh,
    compiler_params=pltpu.CompilerParams(needs_layout_passes=False),
    scratch_types=[
        pltpu.VMEM((DIM, NUM_REL), jnp.float32),
        pltpu.VMEM((2, DIM, CH), jnp.float32),
        pltpu.VMEM((2, DIM, CH), jnp.float32),
        pltpu.VMEM((BPW,), jnp.int32),
        pltpu.VMEM((BPW,), jnp.float32),
        pltpu.SemaphoreType.DMA,
        pltpu.SemaphoreType.DMA,
        pltpu.SemaphoreType.DMA,
    ],
)
def _distmult_sc(hT_hbm, r_hbm, tT_hbm, dT_hbm, out_hbm,
                 d_v, h_v, t_v, r_v, o_v, sem_d, sem0, sem1):
    wid = lax.axis_index("s") * 2 + lax.axis_index("c")
    base = wid * BPW
    sems = [sem0, sem1]

    cp_d = [
        pltpu.async_copy(dT_hbm.at[pl.ds(k * 16, 16)],
                         d_v.at[pl.ds(k * 16, 16)], sem_d)
        for k in range(4)
    ]
    pltpu.sync_copy(r_hbm.at[pl.ds(base, BPW)], r_v)

    def fire(c):
        p = c % 2
        cps = []
        for k in range(2):
            rows = pl.ds(k * 32, 32)
            cps.append(pltpu.async_copy(
                hT_hbm.at[rows, pl.ds(base + c * CH, CH)],
                h_v.at[p].at[rows], sems[p]))
            cps.append(pltpu.async_copy(
                tT_hbm.at[rows, pl.ds(base + c * CH, CH)],
                t_v.at[p].at[rows], sems[p]))
        return cps

    pend = [fire(0), fire(1)]
    for cp in cp_d:
        cp.wait()

    for c in range(NCH):
        p = c % 2
        for cp in pend[c]:
            cp.wait()
        hb = h_v.at[p]
        tb = t_v.at[p]

        def grp(g, carry, c=c, hb=hb, tb=tb):
            ids = r_v[pl.ds(c * CH + g * L, L)]
            z = jnp.zeros((L,), jnp.float32)

            def jblk(b, st):
                a0, a1, jv = st
                for u in range(8):
                    j = b * 8 + u
                    hv = hb[j, pl.ds(g * L, L)]
                    tv = tb[j, pl.ds(g * L, L)]
                    dv = plsc.load_gather(d_v, [jv, ids])
                    pv = (hv * tv) * dv
                    if u % 2 == 0:
                        a0 = a0 + pv
                    else:
                        a1 = a1 + pv
                    jv = jv + 1
                return a0, a1, jv

            a0, a1, _ = lax.fori_loop(
                0, DIM // 8, jblk, (z, z, jnp.zeros((L,), jnp.int32)))
            o_v[pl.ds(c * CH + g * L, L)] = a0 + a1
            return carry

        lax.fori_loop(0, GPC, grp, 0)
        if c + 2 < NCH:
            pend.append(fire(c + 2))

    pltpu.sync_copy(o_v, out_hbm.at[pl.ds(base, BPW)])


def kernel(h, r, t, diag):
    return _distmult_sc(h.T, r.astype(jnp.int32), t.T, diag.T)


# speedup vs baseline: 1.1649x; 1.1649x over previous
import functools

import jax
import jax.numpy as jnp
from jax import lax
from jax.experimental import pallas as pl
from jax.experimental.pallas import tpu as pltpu
from jax.experimental.pallas import tpu_sc as plsc

BATCH = 16384
DIM = 64
NUM_REL = 1000
L = 16
NW = 32
BPW = BATCH // NW
CH = 128
NCH = BPW // CH
GPC = CH // L

_mesh = plsc.VectorSubcoreMesh(core_axis_name="c", subcore_axis_name="s")


@functools.partial(
    pl.kernel,
    out_type=jax.ShapeDtypeStruct((BATCH,), jnp.float32),
    mesh=_mesh,
    compiler_params=pltpu.CompilerParams(needs_layout_passes=False),
    scratch_types=[
        pltpu.VMEM((DIM, NUM_REL), jnp.float32),
        pltpu.VMEM((2, DIM, CH), jnp.float32),
        pltpu.VMEM((2, DIM, CH), jnp.float32),
        pltpu.VMEM((BPW,), jnp.int32),
        pltpu.VMEM((BPW,), jnp.float32),
        pltpu.VMEM_SHARED((DIM, NUM_REL), jnp.float32),
        pltpu.SemaphoreType.DMA,
        pltpu.SemaphoreType.DMA,
        pltpu.SemaphoreType.DMA,
    ],
)
def _distmult_sc(hT_hbm, r_hbm, tT_hbm, dT_hbm, out_hbm,
                 d_v, h_v, t_v, r_v, o_v, d_sh, sem_d, sem0, sem1):
    sid = lax.axis_index("s")
    wid = sid * 2 + lax.axis_index("c")
    base = wid * BPW
    sems = [sem0, sem1]

    @pl.when(sid == 0)
    def _():
        pltpu.sync_copy(dT_hbm, d_sh)

    pltpu.sync_copy(r_hbm.at[pl.ds(base, BPW)], r_v)

    def fire(c):
        p = c % 2
        ha = pltpu.async_copy(
            hT_hbm.at[:, pl.ds(base + c * CH, CH)], h_v.at[p], sems[p])
        ta = pltpu.async_copy(
            tT_hbm.at[:, pl.ds(base + c * CH, CH)], t_v.at[p], sems[p])
        return ha, ta

    pend = [fire(0), fire(1)]
    plsc.subcore_barrier()
    pltpu.sync_copy(d_sh, d_v)

    for c in range(NCH):
        p = c % 2
        for cp in pend[c]:
            cp.wait()
        hb = h_v.at[p]
        tb = t_v.at[p]

        def grp(g, carry, c=c, hb=hb, tb=tb):
            ids = r_v[pl.ds(c * CH + g * L, L)]
            z = jnp.zeros((L,), jnp.float32)

            def jblk(b, st):
                a0, a1, jv = st
                for u in range(8):
                    j = b * 8 + u
                    hv = hb[j, pl.ds(g * L, L)]
                    tv = tb[j, pl.ds(g * L, L)]
                    dv = plsc.load_gather(d_v, [jv, ids])
                    pv = (hv * tv) * dv
                    if u % 2 == 0:
                        a0 = a0 + pv
                    else:
                        a1 = a1 + pv
                    jv = jv + 1
                return a0, a1, jv

            a0, a1, _ = lax.fori_loop(
                0, DIM // 8, jblk, (z, z, jnp.zeros((L,), jnp.int32)))
            o_v[pl.ds(c * CH + g * L, L)] = a0 + a1
            return carry

        lax.fori_loop(0, GPC, grp, 0)
        if c + 2 < NCH:
            pend.append(fire(c + 2))

    pltpu.sync_copy(o_v, out_hbm.at[pl.ds(base, BPW)])


def kernel(h, r, t, diag):
    return _distmult_sc(h.T, r.astype(jnp.int32), t.T, diag.T)


# speedup vs baseline: 1.1672x; 1.0019x over previous
import functools

import jax
import jax.numpy as jnp
from jax import lax
from jax.experimental import pallas as pl
from jax.experimental.pallas import tpu as pltpu
from jax.experimental.pallas import tpu_sc as plsc

BATCH = 16384
DIM = 64
NUM_REL = 1000
L = 16
NW = 32
BPW = BATCH // NW
CH = 128
NCH = BPW // CH
GPC = CH // L

_mesh = plsc.VectorSubcoreMesh(core_axis_name="c", subcore_axis_name="s")


@functools.partial(
    pl.kernel,
    out_type=jax.ShapeDtypeStruct((BATCH,), jnp.float32),
    mesh=_mesh,
    compiler_params=pltpu.CompilerParams(needs_layout_passes=False),
    scratch_types=[
        pltpu.VMEM((DIM, 1024), jnp.float32),
        pltpu.VMEM((2, DIM, CH), jnp.float32),
        pltpu.VMEM((2, DIM, CH), jnp.float32),
        pltpu.VMEM((BPW,), jnp.int32),
        pltpu.VMEM((BPW,), jnp.float32),
        pltpu.VMEM_SHARED((DIM, 1024), jnp.float32),
        pltpu.SemaphoreType.DMA,
        pltpu.SemaphoreType.DMA,
        pltpu.SemaphoreType.DMA,
    ],
)
def _distmult_sc(hT_hbm, r_hbm, tT_hbm, dT_hbm, out_hbm,
                 d_v, h_v, t_v, r_v, o_v, d_sh, sem_d, sem0, sem1):
    sid = lax.axis_index("s")
    wid = sid * 2 + lax.axis_index("c")
    base = wid * BPW
    sems = [sem0, sem1]

    rows4 = pl.ds(sid * 4, 4)
    pltpu.sync_copy(dT_hbm.at[rows4], d_sh.at[rows4])
    pltpu.sync_copy(r_hbm.at[pl.ds(base, BPW)], r_v)

    def fire(c):
        p = c % 2
        ha = pltpu.async_copy(
            hT_hbm.at[:, pl.ds(base + c * CH, CH)], h_v.at[p], sems[p])
        ta = pltpu.async_copy(
            tT_hbm.at[:, pl.ds(base + c * CH, CH)], t_v.at[p], sems[p])
        return ha, ta

    pend = [fire(0), fire(1)]
    plsc.subcore_barrier()
    pltpu.sync_copy(d_sh, d_v)

    for c in range(NCH):
        p = c % 2
        for cp in pend[c]:
            cp.wait()
        hb = h_v.at[p]
        tb = t_v.at[p]

        def grp(g, carry, c=c, hb=hb, tb=tb):
            ids = r_v[pl.ds(c * CH + g * L, L)]
            z = jnp.zeros((L,), jnp.float32)

            def jblk(b, st):
                a0, a1, jv = st
                for u in range(8):
                    j = b * 8 + u
                    hv = hb[j, pl.ds(g * L, L)]
                    tv = tb[j, pl.ds(g * L, L)]
                    dv = plsc.load_gather(d_v, [jv, ids])
                    pv = (hv * tv) * dv
                    if u % 2 == 0:
                        a0 = a0 + pv
                    else:
                        a1 = a1 + pv
                    jv = jv + 1
                return a0, a1, jv

            a0, a1, _ = lax.fori_loop(
                0, DIM // 8, jblk, (z, z, jnp.zeros((L,), jnp.int32)))
            o_v[pl.ds(c * CH + g * L, L)] = a0 + a1
            return carry

        lax.fori_loop(0, GPC, grp, 0)
        if c + 2 < NCH:
            pend.append(fire(c + 2))

    pltpu.sync_copy(o_v, out_hbm.at[pl.ds(base, BPW)])


def kernel(h, r, t, diag):
    dT = jnp.pad(diag.T, ((0, 0), (0, 1024 - NUM_REL)))
    return _distmult_sc(h.T, r.astype(jnp.int32), t.T, dT)
